# baseline (speedup 1.0000x reference)
import jax
import jax.numpy as jnp
from jax.experimental import pallas as pl
from jax.experimental.pallas import tpu as pltpu

B, N = 4, 2048
IN_C, OUT_C, K = 64, 64, 16
D = IN_C + 3
DP = 128
T = 1024
NT = N // T
NSTEP = B * NT


def _select_mask(s):
    h = N // 2
    work = jnp.minimum(s[:, :h], s[:, h:])
    aux = jnp.maximum(s[:, :h], s[:, h:])
    for _ in range(K - 1):
        m = jnp.min(work, axis=1, keepdims=True)
        hit = work == m
        work = jnp.where(hit, aux, work)
        aux = jnp.where(hit, jnp.float32(3e38), aux)
    m = jnp.min(work, axis=1, keepdims=True)
    return (s <= m).astype(jnp.float32)


def _dot(a, b, prec=None):
    return jax.lax.dot_general(a, b, (((1,), (0,)), ((), ())),
                               precision=prec,
                               preferred_element_type=jnp.float32)


def _graphconv(xsplit_ref, xt, s, wrel_ref, wroot_ref, brel_ref):
    maskf = _select_mask(s)
    maskb = maskf.astype(jnp.bfloat16)
    agg = (_dot(maskb, xsplit_ref[0])
           + _dot(maskb, xsplit_ref[1])
           + _dot(maskb, xsplit_ref[2]))
    out = _dot(agg, wrel_ref[...]) + brel_ref[...] + _dot(xt, wroot_ref[...])
    out = jnp.maximum(out, 0.0)
    chan = jax.lax.broadcasted_iota(jnp.int32, (T, DP), 1)
    return jnp.where(chan < 3, xt, out)


def _body(xc_ref, xp_ref, wrel_ref, wroot_ref, brel_ref, out_ref,
          s_scr, sq_scr, xsplit_ref, last):
    i = pl.program_id(0)
    tc = jnp.minimum(i, NSTEP - 1) % NT
    tp = jnp.maximum(i - 1, 0) % NT

    s_old = s_scr[...]
    xt_old = xp_ref[0, pl.ds(tp * T, T), :]
    out = _graphconv(xsplit_ref, xt_old, s_old, wrel_ref, wroot_ref,
                     brel_ref)
    if last is None:
        out_ref[0] = out
    else:
        wloc_ref, bloc_ref = last
        head = _dot(out, wloc_ref[...]) + bloc_ref[...]
        chan = jax.lax.broadcasted_iota(jnp.int32, (T, DP), 1)
        out_ref[0] = jnp.where(chan < 3, xt_old + jnp.tanh(head), out)

    xb = xc_ref[0]

    @pl.when(tc == 0)
    def _():
        ones = jnp.ones((1, DP), jnp.float32)
        sq_scr[...] = jax.lax.dot_general(
            ones, xb * xb, (((1,), (1,)), ((), ())),
            precision=jax.lax.Precision.HIGHEST,
            preferred_element_type=jnp.float32)
        xh = xb.astype(jnp.bfloat16)
        r1 = xb - xh.astype(jnp.float32)
        xm = r1.astype(jnp.bfloat16)
        xl = (r1 - xm.astype(jnp.float32)).astype(jnp.bfloat16)
        xsplit_ref[0] = xh
        xsplit_ref[1] = xm
        xsplit_ref[2] = xl

    xt_new = xc_ref[0, pl.ds(tc * T, T), :]
    g = jax.lax.dot_general(xt_new, xb, (((1,), (1,)), ((), ())),
                            preferred_element_type=jnp.float32)
    s_new = sq_scr[...] - 2.0 * g
    col = jax.lax.broadcasted_iota(jnp.int32, (T, N), 1)
    row = jax.lax.broadcasted_iota(jnp.int32, (T, N), 0) + tc * T
    s_scr[...] = jnp.where(col == row, jnp.float32(3e38), s_new)


def _layer_kernel(xc_ref, xp_ref, wrel_ref, wroot_ref, brel_ref, out_ref,
                  s_scr, sq_scr, xsplit_ref):
    _body(xc_ref, xp_ref, wrel_ref, wroot_ref, brel_ref, out_ref,
          s_scr, sq_scr, xsplit_ref, None)


def _last_kernel(xc_ref, xp_ref, wrel_ref, wroot_ref, brel_ref, wloc_ref,
                 bloc_ref, out_ref, s_scr, sq_scr, xsplit_ref):
    _body(xc_ref, xp_ref, wrel_ref, wroot_ref, brel_ref, out_ref,
          s_scr, sq_scr, xsplit_ref, (wloc_ref, bloc_ref))


def _call(kernel_fn, x, *weights):
    in_specs = [
        pl.BlockSpec((1, N, DP),
                     lambda i: (jnp.minimum(i, NSTEP - 1) // NT, 0, 0)),
        pl.BlockSpec((1, N, DP),
                     lambda i: (jnp.maximum(i - 1, 0) // NT, 0, 0)),
    ]
    in_specs += [pl.BlockSpec(w.shape, lambda i: (0,) * w.ndim)
                 for w in weights]
    return pl.pallas_call(
        kernel_fn,
        grid=(NSTEP + 1,),
        in_specs=in_specs,
        out_specs=pl.BlockSpec(
            (1, T, DP),
            lambda i: (jnp.maximum(i - 1, 0) // NT,
                       jnp.maximum(i - 1, 0) % NT, 0)),
        out_shape=jax.ShapeDtypeStruct((B, N, DP), jnp.float32),
        scratch_shapes=[
            pltpu.VMEM((T, N), jnp.float32),
            pltpu.VMEM((1, N), jnp.float32),
            pltpu.VMEM((3, N, DP), jnp.bfloat16),
        ],
    )(x, x, *weights)


def _pad_rel(w):
    return jnp.zeros((DP, DP), jnp.float32).at[:D, 3:3 + OUT_C].set(w)


def _pad_b(b):
    return jnp.zeros((1, DP), jnp.float32).at[0, 3:3 + OUT_C].set(b)


def kernel(pcd_location, pcd_features, W_rel0, b_rel0, W_root0,
           W_rel1, b_rel1, W_root1, W_rel2, b_rel2, W_root2, W_loc, b_loc):
    loc_t = jnp.transpose(pcd_location, (0, 2, 1))
    feat_t = jnp.transpose(pcd_features, (0, 2, 1))
    x = jnp.concatenate(
        [loc_t, feat_t, jnp.zeros((B, N, DP - D), jnp.float32)], axis=2)

    x = _call(_layer_kernel, x,
              _pad_rel(W_rel0), _pad_rel(W_root0), _pad_b(b_rel0))
    x = _call(_layer_kernel, x,
              _pad_rel(W_rel1), _pad_rel(W_root1), _pad_b(b_rel1))

    wloc = jnp.zeros((DP, DP), jnp.float32).at[:D, 0:3].set(W_loc.T)
    bloc = jnp.zeros((1, DP), jnp.float32).at[0, 0:3].set(b_loc)
    x = _call(_last_kernel, x,
              _pad_rel(W_rel2), _pad_rel(W_root2), _pad_b(b_rel2),
              wloc, bloc)

    loc_out = jnp.transpose(x[:, :, 0:3], (0, 2, 1))
    feat_out = jnp.transpose(x[:, :, 3:3 + OUT_C], (0, 2, 1))
    return (loc_out, feat_out)


# speedup vs baseline: 1.0713x; 1.0713x over previous
import jax
import jax.numpy as jnp
from jax.experimental import pallas as pl
from jax.experimental.pallas import tpu as pltpu

B, N = 4, 2048
IN_C, OUT_C, K = 64, 64, 16
D = IN_C + 3
DP = 128
T = 512
NT = N // T
NSTEP = B * NT


def _select_mask(s):
    h = N // 2
    work = jnp.minimum(s[:, :h], s[:, h:])
    aux = jnp.maximum(s[:, :h], s[:, h:])
    for _ in range(K - 1):
        m = jnp.min(work, axis=1, keepdims=True)
        hit = work == m
        work = jnp.where(hit, aux, work)
        aux = jnp.where(hit, jnp.float32(3e38), aux)
    m = jnp.min(work, axis=1, keepdims=True)
    return (s <= m).astype(jnp.float32)


def _dot(a, b, prec=None):
    return jax.lax.dot_general(a, b, (((1,), (0,)), ((), ())),
                               precision=prec,
                               preferred_element_type=jnp.float32)


def _graphconv(xsplit_ref, xt, s, wrel_ref, wroot_ref, brel_ref):
    maskf = _select_mask(s)
    maskb = maskf.astype(jnp.bfloat16)
    agg = (_dot(maskb, xsplit_ref[0])
           + _dot(maskb, xsplit_ref[1])
           + _dot(maskb, xsplit_ref[2]))
    out = _dot(agg, wrel_ref[...]) + brel_ref[...] + _dot(xt, wroot_ref[...])
    out = jnp.maximum(out, 0.0)
    chan = jax.lax.broadcasted_iota(jnp.int32, (T, DP), 1)
    return jnp.where(chan < 3, xt, out)


def _body(xc_ref, xp_ref, wrel_ref, wroot_ref, brel_ref, out_ref,
          s_scr, sq_scr, xsplit_ref, last):
    i = pl.program_id(0)
    tc = jnp.minimum(i, NSTEP - 1) % NT
    tp = jnp.maximum(i - 1, 0) % NT

    s_old = s_scr[...]
    xt_old = xp_ref[0, pl.ds(tp * T, T), :]
    out = _graphconv(xsplit_ref, xt_old, s_old, wrel_ref, wroot_ref,
                     brel_ref)
    if last is None:
        out_ref[0] = out
    else:
        wloc_ref, bloc_ref = last
        head = _dot(out, wloc_ref[...]) + bloc_ref[...]
        chan = jax.lax.broadcasted_iota(jnp.int32, (T, DP), 1)
        out_ref[0] = jnp.where(chan < 3, xt_old + jnp.tanh(head), out)

    xb = xc_ref[0]

    @pl.when(tc == 0)
    def _():
        ones = jnp.ones((1, DP), jnp.float32)
        sq_scr[...] = jax.lax.dot_general(
            ones, xb * xb, (((1,), (1,)), ((), ())),
            precision=jax.lax.Precision.HIGHEST,
            preferred_element_type=jnp.float32)
        xh = xb.astype(jnp.bfloat16)
        r1 = xb - xh.astype(jnp.float32)
        xm = r1.astype(jnp.bfloat16)
        xl = (r1 - xm.astype(jnp.float32)).astype(jnp.bfloat16)
        xsplit_ref[0] = xh
        xsplit_ref[1] = xm
        xsplit_ref[2] = xl

    xt_new = xc_ref[0, pl.ds(tc * T, T), :]
    g = jax.lax.dot_general(xt_new, xb, (((1,), (1,)), ((), ())),
                            preferred_element_type=jnp.float32)
    s_new = sq_scr[...] - 2.0 * g
    col = jax.lax.broadcasted_iota(jnp.int32, (T, N), 1)
    row = jax.lax.broadcasted_iota(jnp.int32, (T, N), 0) + tc * T
    s_scr[...] = jnp.where(col == row, jnp.float32(3e38), s_new)


def _layer_kernel(xc_ref, xp_ref, wrel_ref, wroot_ref, brel_ref, out_ref,
                  s_scr, sq_scr, xsplit_ref):
    _body(xc_ref, xp_ref, wrel_ref, wroot_ref, brel_ref, out_ref,
          s_scr, sq_scr, xsplit_ref, None)


def _last_kernel(xc_ref, xp_ref, wrel_ref, wroot_ref, brel_ref, wloc_ref,
                 bloc_ref, out_ref, s_scr, sq_scr, xsplit_ref):
    _body(xc_ref, xp_ref, wrel_ref, wroot_ref, brel_ref, out_ref,
          s_scr, sq_scr, xsplit_ref, (wloc_ref, bloc_ref))


def _call(kernel_fn, x, *weights):
    in_specs = [
        pl.BlockSpec((1, N, DP),
                     lambda i: (jnp.minimum(i, NSTEP - 1) // NT, 0, 0)),
        pl.BlockSpec((1, N, DP),
                     lambda i: (jnp.maximum(i - 1, 0) // NT, 0, 0)),
    ]
    in_specs += [pl.BlockSpec(w.shape, lambda i: (0,) * w.ndim)
                 for w in weights]
    return pl.pallas_call(
        kernel_fn,
        grid=(NSTEP + 1,),
        in_specs=in_specs,
        out_specs=pl.BlockSpec(
            (1, T, DP),
            lambda i: (jnp.maximum(i - 1, 0) // NT,
                       jnp.maximum(i - 1, 0) % NT, 0)),
        out_shape=jax.ShapeDtypeStruct((B, N, DP), jnp.float32),
        scratch_shapes=[
            pltpu.VMEM((T, N), jnp.float32),
            pltpu.VMEM((1, N), jnp.float32),
            pltpu.VMEM((3, N, DP), jnp.bfloat16),
        ],
    )(x, x, *weights)


def _pad_rel(w):
    return jnp.zeros((DP, DP), jnp.float32).at[:D, 3:3 + OUT_C].set(w)


def _pad_b(b):
    return jnp.zeros((1, DP), jnp.float32).at[0, 3:3 + OUT_C].set(b)


def kernel(pcd_location, pcd_features, W_rel0, b_rel0, W_root0,
           W_rel1, b_rel1, W_root1, W_rel2, b_rel2, W_root2, W_loc, b_loc):
    loc_t = jnp.transpose(pcd_location, (0, 2, 1))
    feat_t = jnp.transpose(pcd_features, (0, 2, 1))
    x = jnp.concatenate(
        [loc_t, feat_t, jnp.zeros((B, N, DP - D), jnp.float32)], axis=2)

    x = _call(_layer_kernel, x,
              _pad_rel(W_rel0), _pad_rel(W_root0), _pad_b(b_rel0))
    x = _call(_layer_kernel, x,
              _pad_rel(W_rel1), _pad_rel(W_root1), _pad_b(b_rel1))

    wloc = jnp.zeros((DP, DP), jnp.float32).at[:D, 0:3].set(W_loc.T)
    bloc = jnp.zeros((1, DP), jnp.float32).at[0, 0:3].set(b_loc)
    x = _call(_last_kernel, x,
              _pad_rel(W_rel2), _pad_rel(W_root2), _pad_b(b_rel2),
              wloc, bloc)

    loc_out = jnp.transpose(x[:, :, 0:3], (0, 2, 1))
    feat_out = jnp.transpose(x[:, :, 3:3 + OUT_C], (0, 2, 1))
    return (loc_out, feat_out)
